# Initial kernel scaffold; baseline (speedup 1.0000x reference)
import jax
import jax.numpy as jnp
from jax.experimental import pallas as pl


def kernel(V, data, g_W1, g_b1, g_W2, g_b2, fs_W1, fs_b1, fs_W2, fs_b2):
    raise NotImplementedError("write your pallas kernel here")


# speedup vs baseline: 5.6344x; 5.6344x over previous
import functools

import jax
import jax.numpy as jnp
from jax import lax
from jax.experimental import pallas as pl
from jax.experimental.pallas import tpu as pltpu

B = 2
SEQ = 2048
EMB = 1024
HID = 1024
NW = 11
NL = 12
OFFSETS = (0,) + tuple(2 ** k for k in range(NL - 1))

_PREC = lax.Precision.HIGHEST


def _gelu(x):
    return 0.5 * x * (1.0 + lax.erf(x * 0.7071067811865476))


def _w_body(x_ref, w1_ref, b1_ref, w2_ref, b2_ref, out_ref):
    h = jnp.dot(x_ref[...], w1_ref[0], preferred_element_type=jnp.float32,
                precision=_PREC)
    h = _gelu(h + b1_ref[0, 0])
    w = jnp.dot(h, w2_ref[0], preferred_element_type=jnp.float32,
                precision=_PREC)
    out_ref[0] = w + b2_ref[0, 0]


def _compute_w(data2d, fs_W1, fs_b1, fs_W2, fs_b2, ts):
    n = B * SEQ
    grid = (NW, n // ts)
    return pl.pallas_call(
        _w_body,
        grid=grid,
        in_specs=[
            pl.BlockSpec((ts, EMB), lambda m, s: (s, 0)),
            pl.BlockSpec((1, EMB, HID), lambda m, s: (m, 0, 0)),
            pl.BlockSpec((1, 1, HID), lambda m, s: (m, 0, 0)),
            pl.BlockSpec((1, HID, NL), lambda m, s: (m, 0, 0)),
            pl.BlockSpec((1, 1, NL), lambda m, s: (m, 0, 0)),
        ],
        out_specs=pl.BlockSpec((1, ts, NL), lambda m, s: (m, s, 0)),
        out_shape=jax.ShapeDtypeStruct((NW, n, NL), jnp.float32),
    )(data2d, fs_W1, fs_b1.reshape(NW, 1, HID), fs_W2,
      fs_b2.reshape(NW, 1, NL))


def _gmlp_body(x_ref, w1_ref, b1_ref, w2_ref, b2_ref, out_ref):
    h = jnp.dot(x_ref[...], w1_ref[...], preferred_element_type=jnp.float32,
                precision=_PREC)
    h = _gelu(h + b1_ref[0])
    v = jnp.dot(h, w2_ref[...], preferred_element_type=jnp.float32,
                precision=_PREC)
    out_ref[...] = v + b2_ref[0]


def _gmlp(v2d, g_W1, g_b1, g_W2, g_b2, ts):
    n = B * SEQ
    return pl.pallas_call(
        _gmlp_body,
        grid=(n // ts,),
        in_specs=[
            pl.BlockSpec((ts, EMB), lambda s: (s, 0)),
            pl.BlockSpec((EMB, HID), lambda s: (0, 0)),
            pl.BlockSpec((1, HID), lambda s: (0, 0)),
            pl.BlockSpec((HID, EMB), lambda s: (0, 0)),
            pl.BlockSpec((1, EMB), lambda s: (0, 0)),
        ],
        out_specs=pl.BlockSpec((ts, EMB), lambda s: (s, 0)),
        out_shape=jax.ShapeDtypeStruct((n, EMB), jnp.float32),
    )(v2d, g_W1, g_b1.reshape(1, HID), g_W2, g_b2.reshape(1, EMB))


def _chain_body(v_ref, w_ref, out_ref, scratch):
    m = pl.program_id(1)

    @pl.when(m == 0)
    def _init():
        scratch[...] = v_ref[...]

    x = scratch[...]
    w = w_ref[0]
    acc = x + w[:, :, 0:1] * x
    for l, off in enumerate(OFFSETS[1:], start=1):
        rolled = jnp.concatenate([x[:, off:, :], x[:, :off, :]], axis=1)
        acc = acc + w[:, :, l:l + 1] * rolled
    scratch[...] = acc

    @pl.when(m == NW - 1)
    def _store():
        out_ref[...] = acc


def _chain(v1, w, te):
    grid = (EMB // te, NW)
    return pl.pallas_call(
        _chain_body,
        grid=grid,
        in_specs=[
            pl.BlockSpec((B, SEQ, te), lambda e, m: (0, 0, e)),
            pl.BlockSpec((1, B, SEQ, NL), lambda e, m: (m, 0, 0, 0)),
        ],
        out_specs=pl.BlockSpec((B, SEQ, te), lambda e, m: (0, 0, e)),
        out_shape=jax.ShapeDtypeStruct((B, SEQ, EMB), jnp.float32),
        scratch_shapes=[pltpu.VMEM((B, SEQ, te), jnp.float32)],
    )(v1, w)


@functools.partial(jax.jit, static_argnames=())
def kernel(V, data, g_W1, g_b1, g_W2, g_b2, fs_W1, fs_b1, fs_W2, fs_b2):
    data2d = data.reshape(B * SEQ, EMB)
    v2d = V.reshape(B * SEQ, EMB)
    w = _compute_w(data2d, fs_W1, fs_b1, fs_W2, fs_b2, ts=1024)
    w = w.reshape(NW, B, SEQ, NL)
    v1 = _gmlp(v2d, g_W1, g_b1, g_W2, g_b2, ts=1024).reshape(B, SEQ, EMB)
    return _chain(v1, w, te=256)


# speedup vs baseline: 8.8006x; 1.5619x over previous
import functools

import jax
import jax.numpy as jnp
from jax import lax
from jax.experimental import pallas as pl
from jax.experimental.pallas import tpu as pltpu

B = 2
SEQ = 2048
EMB = 1024
HID = 1024
NW = 11
NL = 12
OFFSETS = (0,) + tuple(2 ** k for k in range(NL - 1))

def _dot3(x, w):
    xh = x.astype(jnp.bfloat16)
    xl = (x - xh.astype(jnp.float32)).astype(jnp.bfloat16)
    wh = w.astype(jnp.bfloat16)
    wl = (w - wh.astype(jnp.float32)).astype(jnp.bfloat16)
    d = lambda a, b: jnp.dot(a, b, preferred_element_type=jnp.float32)
    return d(xh, wh) + (d(xh, wl) + d(xl, wh))


def _gelu(x):
    return 0.5 * x * (1.0 + lax.erf(x * 0.7071067811865476))


def _w_body(x_ref, w1_ref, b1_ref, w2_ref, b2_ref, out_ref):
    h = _dot3(x_ref[...], w1_ref[0])
    h = _gelu(h + b1_ref[0, 0])
    w = _dot3(h, w2_ref[0])
    out_ref[0] = w + b2_ref[0, 0]


def _compute_w(data2d, fs_W1, fs_b1, fs_W2, fs_b2, ts):
    n = B * SEQ
    grid = (NW, n // ts)
    return pl.pallas_call(
        _w_body,
        grid=grid,
        in_specs=[
            pl.BlockSpec((ts, EMB), lambda m, s: (s, 0)),
            pl.BlockSpec((1, EMB, HID), lambda m, s: (m, 0, 0)),
            pl.BlockSpec((1, 1, HID), lambda m, s: (m, 0, 0)),
            pl.BlockSpec((1, HID, NL), lambda m, s: (m, 0, 0)),
            pl.BlockSpec((1, 1, NL), lambda m, s: (m, 0, 0)),
        ],
        out_specs=pl.BlockSpec((1, ts, NL), lambda m, s: (m, s, 0)),
        out_shape=jax.ShapeDtypeStruct((NW, n, NL), jnp.float32),
    )(data2d, fs_W1, fs_b1.reshape(NW, 1, HID), fs_W2,
      fs_b2.reshape(NW, 1, NL))


def _gmlp_body(x_ref, w1_ref, b1_ref, w2_ref, b2_ref, out_ref):
    h = _dot3(x_ref[...], w1_ref[...])
    h = _gelu(h + b1_ref[0])
    v = _dot3(h, w2_ref[...])
    out_ref[...] = v + b2_ref[0]


def _gmlp(v2d, g_W1, g_b1, g_W2, g_b2, ts):
    n = B * SEQ
    return pl.pallas_call(
        _gmlp_body,
        grid=(n // ts,),
        in_specs=[
            pl.BlockSpec((ts, EMB), lambda s: (s, 0)),
            pl.BlockSpec((EMB, HID), lambda s: (0, 0)),
            pl.BlockSpec((1, HID), lambda s: (0, 0)),
            pl.BlockSpec((HID, EMB), lambda s: (0, 0)),
            pl.BlockSpec((1, EMB), lambda s: (0, 0)),
        ],
        out_specs=pl.BlockSpec((ts, EMB), lambda s: (s, 0)),
        out_shape=jax.ShapeDtypeStruct((n, EMB), jnp.float32),
    )(v2d, g_W1, g_b1.reshape(1, HID), g_W2, g_b2.reshape(1, EMB))


def _chain_body(v_ref, w_ref, out_ref, scratch):
    m = pl.program_id(1)

    @pl.when(m == 0)
    def _init():
        scratch[...] = v_ref[...]

    x = scratch[...]
    w = w_ref[0]
    acc = x + w[:, :, 0:1] * x
    for l, off in enumerate(OFFSETS[1:], start=1):
        rolled = jnp.concatenate([x[:, off:, :], x[:, :off, :]], axis=1)
        acc = acc + w[:, :, l:l + 1] * rolled
    scratch[...] = acc

    @pl.when(m == NW - 1)
    def _store():
        out_ref[...] = acc


def _chain(v1, w, te):
    grid = (EMB // te, NW)
    return pl.pallas_call(
        _chain_body,
        grid=grid,
        in_specs=[
            pl.BlockSpec((B, SEQ, te), lambda e, m: (0, 0, e)),
            pl.BlockSpec((1, B, SEQ, NL), lambda e, m: (m, 0, 0, 0)),
        ],
        out_specs=pl.BlockSpec((B, SEQ, te), lambda e, m: (0, 0, e)),
        out_shape=jax.ShapeDtypeStruct((B, SEQ, EMB), jnp.float32),
        scratch_shapes=[pltpu.VMEM((B, SEQ, te), jnp.float32)],
    )(v1, w)


@functools.partial(jax.jit, static_argnames=())
def kernel(V, data, g_W1, g_b1, g_W2, g_b2, fs_W1, fs_b1, fs_W2, fs_b2):
    data2d = data.reshape(B * SEQ, EMB)
    v2d = V.reshape(B * SEQ, EMB)
    w = _compute_w(data2d, fs_W1, fs_b1, fs_W2, fs_b2, ts=1024)
    w = w.reshape(NW, B, SEQ, NL)
    v1 = _gmlp(v2d, g_W1, g_b1, g_W2, g_b2, ts=1024).reshape(B, SEQ, EMB)
    return _chain(v1, w, te=256)
